# Initial kernel scaffold; baseline (speedup 1.0000x reference)
import jax
import jax.numpy as jnp
from jax.experimental import pallas as pl


def kernel(code_ids_batch, table):
    raise NotImplementedError("write your pallas kernel here")


# speedup vs baseline: 5.5434x; 5.5434x over previous
import functools

import jax
import jax.numpy as jnp
from jax import lax
from jax.experimental import pallas as pl
from jax.experimental.pallas import tpu as pltpu
from jax.experimental.pallas import tpu_sc as plsc

NUM_CODES = 1000000
DIM = 16
PAD_IDX = 0
BATCH = 16384
HIST = 200

NC = 2
NS = 16
NW = NC * NS
ROWS_PER_WORKER = BATCH // NW
CHUNK = 16
NCHUNK = ROWS_PER_WORKER // CHUNK
ENTRIES = CHUNK * HIST
GBURST = 128
NGATHER = ENTRIES // GBURST


def _sc_body(ids1d_hbm, table_hbm, out_hbm,
             idx1, emb, outb, t0v, gsem):
    wid = lax.axis_index("s") * NC + lax.axis_index("c")

    pltpu.sync_copy(table_hbm.at[pl.ds(0, 1)], t0v)
    t0 = t0v[0]

    lane = lax.iota(jnp.int32, 16)
    hi_mask = lane >= 8

    def chunk_body(c, carry):
        base = wid * ROWS_PER_WORKER + c * CHUNK
        ebase = base * HIST

        pltpu.sync_copy(ids1d_hbm.at[pl.ds(ebase, ENTRIES)], idx1)

        copies = []
        for g in range(NGATHER):
            copies.append(pltpu.async_copy(
                table_hbm.at[idx1.at[pl.ds(g * GBURST, GBURST)]],
                emb.at[pl.ds(g * GBURST, GBURST)],
                gsem))
        for cp in copies:
            cp.wait()

        for r in range(CHUNK):
            pz = jnp.zeros((16,), jnp.int32)
            for j in range(12):
                v = idx1[pl.ds(r * HIST + j * 16, 16)]
                pz = pz + (v == PAD_IDX).astype(jnp.int32)
            v = idx1[pl.ds(r * HIST + HIST - 16, 16)]
            pz = pz + ((v == PAD_IDX) & hi_mask).astype(jnp.int32)
            npad = jnp.sum(pz)

            def acc_body(j, a):
                return a + emb[r * HIST + j]
            acc = lax.fori_loop(0, HIST, acc_body,
                                jnp.zeros((DIM,), jnp.float32))

            cntv = jnp.broadcast_to((HIST - npad).astype(jnp.float32), (DIM,))
            invv = jnp.float32(1.0) / jnp.maximum(cntv, jnp.float32(1.0))
            res = (acc - npad.astype(jnp.float32) * t0) * invv
            outb[r] = jnp.where(cntv > 0, res, jnp.float32(0.0))

        pltpu.sync_copy(outb, out_hbm.at[pl.ds(base, CHUNK)])
        return carry

    lax.fori_loop(0, NCHUNK, chunk_body, 0)


@jax.jit
def _visit_encode(ids, table):
    mesh = plsc.VectorSubcoreMesh(core_axis_name="c", subcore_axis_name="s")
    ker = pl.kernel(
        _sc_body,
        out_type=jax.ShapeDtypeStruct((BATCH, DIM), jnp.float32),
        mesh=mesh,
        scratch_types=[
            pltpu.VMEM((ENTRIES,), jnp.int32),
            pltpu.VMEM((ENTRIES, DIM), jnp.float32),
            pltpu.VMEM((CHUNK, DIM), jnp.float32),
            pltpu.VMEM((1, DIM), jnp.float32),
            pltpu.SemaphoreType.DMA,
        ],
        compiler_params=pltpu.CompilerParams(use_tc_tiling_on_sc=False,
                                             needs_layout_passes=False),
    )
    ids1d = ids.reshape(-1)
    return ker(ids1d, table)


def kernel(code_ids_batch, table):
    ids = code_ids_batch.astype(jnp.int32)
    return _visit_encode(ids, table)


# speedup vs baseline: 8.1931x; 1.4780x over previous
import functools

import jax
import jax.numpy as jnp
from jax import lax
from jax.experimental import pallas as pl
from jax.experimental.pallas import tpu as pltpu
from jax.experimental.pallas import tpu_sc as plsc

NUM_CODES = 1000000
DIM = 16
PAD_IDX = 0
BATCH = 16384
HIST = 200

NC = 2
NS = 16
NW = NC * NS
ROWS_PER_WORKER = BATCH // NW
CHUNK = 16
NCHUNK = ROWS_PER_WORKER // CHUNK
ENTRIES = CHUNK * HIST
GBURST = 128
NGATHER = ENTRIES // GBURST


def _sc_body(ids1d_hbm, table_hbm, out_hbm,
             idx1, emb, outb, t0v, gsem):
    wid = lax.axis_index("s") * NC + lax.axis_index("c")

    pltpu.sync_copy(table_hbm.at[pl.ds(0, 1)], t0v)
    t0 = t0v[0]

    lane = lax.iota(jnp.int32, 16)
    hi_mask = lane >= 8

    def chunk_body(c, carry):
        base = wid * ROWS_PER_WORKER + c * CHUNK
        ebase = base * HIST

        pltpu.sync_copy(ids1d_hbm.at[pl.ds(ebase, ENTRIES)], idx1)

        copies = []
        for g in range(NGATHER):
            copies.append(pltpu.async_copy(
                table_hbm.at[idx1.at[pl.ds(g * GBURST, GBURST)]],
                emb.at[pl.ds(g * GBURST, GBURST)],
                gsem))
        for cp in copies:
            cp.wait()

        for r in range(CHUNK):
            pz = jnp.zeros((16,), jnp.int32)
            for j in range(12):
                v = idx1[pl.ds(r * HIST + j * 16, 16)]
                pz = pz + (v == PAD_IDX).astype(jnp.int32)
            v = idx1[pl.ds(r * HIST + HIST - 16, 16)]
            pz = pz + ((v == PAD_IDX) & hi_mask).astype(jnp.int32)
            npad = jnp.sum(pz)

            def acc_body(j, accs):
                base8 = r * HIST + j * 8
                return tuple(a + emb[base8 + k] for k, a in enumerate(accs))
            accs = lax.fori_loop(
                0, HIST // 8, acc_body,
                tuple(jnp.zeros((DIM,), jnp.float32) for _ in range(8)))
            a0 = (accs[0] + accs[1]) + (accs[2] + accs[3])
            a1 = (accs[4] + accs[5]) + (accs[6] + accs[7])
            acc = a0 + a1

            cntv = jnp.broadcast_to((HIST - npad).astype(jnp.float32), (DIM,))
            invv = jnp.float32(1.0) / jnp.maximum(cntv, jnp.float32(1.0))
            res = (acc - npad.astype(jnp.float32) * t0) * invv
            outb[r] = jnp.where(cntv > 0, res, jnp.float32(0.0))

        pltpu.sync_copy(outb, out_hbm.at[pl.ds(base, CHUNK)])
        return carry

    lax.fori_loop(0, NCHUNK, chunk_body, 0)


@jax.jit
def _visit_encode(ids, table):
    mesh = plsc.VectorSubcoreMesh(core_axis_name="c", subcore_axis_name="s")
    ker = pl.kernel(
        _sc_body,
        out_type=jax.ShapeDtypeStruct((BATCH, DIM), jnp.float32),
        mesh=mesh,
        scratch_types=[
            pltpu.VMEM((ENTRIES,), jnp.int32),
            pltpu.VMEM((ENTRIES, DIM), jnp.float32),
            pltpu.VMEM((CHUNK, DIM), jnp.float32),
            pltpu.VMEM((1, DIM), jnp.float32),
            pltpu.SemaphoreType.DMA,
        ],
        compiler_params=pltpu.CompilerParams(use_tc_tiling_on_sc=False,
                                             needs_layout_passes=False),
    )
    ids1d = ids.reshape(-1)
    return ker(ids1d, table)


def kernel(code_ids_batch, table):
    ids = code_ids_batch.astype(jnp.int32)
    return _visit_encode(ids, table)


# speedup vs baseline: 8.1959x; 1.0003x over previous
import functools

import jax
import jax.numpy as jnp
from jax import lax
from jax.experimental import pallas as pl
from jax.experimental.pallas import tpu as pltpu
from jax.experimental.pallas import tpu_sc as plsc

NUM_CODES = 1000000
DIM = 16
PAD_IDX = 0
BATCH = 16384
HIST = 200

NC = 2
NS = 16
NW = NC * NS
ROWS_PER_WORKER = BATCH // NW
CHUNK = 16
NCHUNK = ROWS_PER_WORKER // CHUNK
ENTRIES = CHUNK * HIST
GBURST = 128
NGATHER = ENTRIES // GBURST


def _sc_body(ids1d_hbm, table_hbm, out_hbm,
             idx1, emb, outb, t0v, gsem):
    wid = lax.axis_index("s") * NC + lax.axis_index("c")

    pltpu.sync_copy(table_hbm.at[pl.ds(0, 1)], t0v)
    t0 = t0v[0]

    lane = lax.iota(jnp.int32, 16)
    hi_mask = lane >= 8

    def chunk_body(c, carry):
        base = wid * ROWS_PER_WORKER + c * CHUNK
        ebase = base * HIST

        pltpu.sync_copy(ids1d_hbm.at[pl.ds(ebase, ENTRIES)], idx1)

        pltpu.async_copy(table_hbm.at[idx1], emb, gsem).wait()

        for r in range(CHUNK):
            pz = jnp.zeros((16,), jnp.int32)
            for j in range(12):
                v = idx1[pl.ds(r * HIST + j * 16, 16)]
                pz = pz + (v == PAD_IDX).astype(jnp.int32)
            v = idx1[pl.ds(r * HIST + HIST - 16, 16)]
            pz = pz + ((v == PAD_IDX) & hi_mask).astype(jnp.int32)
            npad = jnp.sum(pz)

            def acc_body(j, accs):
                base8 = r * HIST + j * 8
                return tuple(a + emb[base8 + k] for k, a in enumerate(accs))
            accs = lax.fori_loop(
                0, HIST // 8, acc_body,
                tuple(jnp.zeros((DIM,), jnp.float32) for _ in range(8)))
            a0 = (accs[0] + accs[1]) + (accs[2] + accs[3])
            a1 = (accs[4] + accs[5]) + (accs[6] + accs[7])
            acc = a0 + a1

            cntv = jnp.broadcast_to((HIST - npad).astype(jnp.float32), (DIM,))
            invv = jnp.float32(1.0) / jnp.maximum(cntv, jnp.float32(1.0))
            res = (acc - npad.astype(jnp.float32) * t0) * invv
            outb[r] = jnp.where(cntv > 0, res, jnp.float32(0.0))

        pltpu.sync_copy(outb, out_hbm.at[pl.ds(base, CHUNK)])
        return carry

    lax.fori_loop(0, NCHUNK, chunk_body, 0)


@jax.jit
def _visit_encode(ids, table):
    mesh = plsc.VectorSubcoreMesh(core_axis_name="c", subcore_axis_name="s")
    ker = pl.kernel(
        _sc_body,
        out_type=jax.ShapeDtypeStruct((BATCH, DIM), jnp.float32),
        mesh=mesh,
        scratch_types=[
            pltpu.VMEM((ENTRIES,), jnp.int32),
            pltpu.VMEM((ENTRIES, DIM), jnp.float32),
            pltpu.VMEM((CHUNK, DIM), jnp.float32),
            pltpu.VMEM((1, DIM), jnp.float32),
            pltpu.SemaphoreType.DMA,
        ],
        compiler_params=pltpu.CompilerParams(use_tc_tiling_on_sc=False,
                                             needs_layout_passes=False),
    )
    ids1d = ids.reshape(-1)
    return ker(ids1d, table)


def kernel(code_ids_batch, table):
    ids = code_ids_batch.astype(jnp.int32)
    return _visit_encode(ids, table)


# speedup vs baseline: 8.2022x; 1.0008x over previous
import jax
import jax.numpy as jnp
from jax import lax
from jax.experimental import pallas as pl
from jax.experimental.pallas import tpu as pltpu
from jax.experimental.pallas import tpu_sc as plsc

NUM_CODES = 1000000
DIM = 16
PAD_IDX = 0
BATCH = 16384
HIST = 200

NC = 2
NS = 16
NW = NC * NS
ROWS_PER_WORKER = BATCH // NW
CHUNK = 16
NCHUNK = ROWS_PER_WORKER // CHUNK


def _sc_body(ids_hbm, table_hbm, out_hbm, idx2, emb3, outb, t0v, gsem):
    wid = lax.axis_index("s") * NC + lax.axis_index("c")

    pltpu.sync_copy(table_hbm.at[pl.ds(0, 1)], t0v)
    t0 = t0v[0]

    lane = lax.iota(jnp.int32, 16)
    hi_mask = lane >= 8

    def chunk_body(c, carry):
        base = wid * ROWS_PER_WORKER + c * CHUNK

        pltpu.sync_copy(ids_hbm.at[pl.ds(base, CHUNK)], idx2)
        copies = [pltpu.async_copy(table_hbm.at[idx2.at[r]],
                                   emb3.at[r], gsem)
                  for r in range(CHUNK)]
        for cp in copies:
            cp.wait()

        for r in range(CHUNK):
            pz = jnp.zeros((16,), jnp.int32)
            for j in range(12):
                v = idx2[r, pl.ds(j * 16, 16)]
                pz = pz + (v == PAD_IDX).astype(jnp.int32)
            v = idx2[r, pl.ds(HIST - 16, 16)]
            pz = pz + ((v == PAD_IDX) & hi_mask).astype(jnp.int32)
            npad = jnp.sum(pz)

            def acc_body(j, accs):
                base8 = j * 8
                return tuple(a + emb3[r, base8 + k]
                             for k, a in enumerate(accs))
            accs = lax.fori_loop(
                0, HIST // 8, acc_body,
                tuple(jnp.zeros((DIM,), jnp.float32) for _ in range(8)))
            a0 = (accs[0] + accs[1]) + (accs[2] + accs[3])
            a1 = (accs[4] + accs[5]) + (accs[6] + accs[7])
            acc = a0 + a1

            cntv = jnp.broadcast_to((HIST - npad).astype(jnp.float32), (DIM,))
            invv = jnp.float32(1.0) / jnp.maximum(cntv, jnp.float32(1.0))
            res = (acc - npad.astype(jnp.float32) * t0) * invv
            outb[r] = jnp.where(cntv > 0, res, jnp.float32(0.0))

        pltpu.sync_copy(outb, out_hbm.at[pl.ds(base, CHUNK)])
        return carry

    lax.fori_loop(0, NCHUNK, chunk_body, 0)


@jax.jit
def _visit_encode(ids, table):
    mesh = plsc.VectorSubcoreMesh(core_axis_name="c", subcore_axis_name="s")
    ker = pl.kernel(
        _sc_body,
        out_type=jax.ShapeDtypeStruct((BATCH, DIM), jnp.float32),
        mesh=mesh,
        scratch_types=[
            pltpu.VMEM((CHUNK, HIST), jnp.int32),
            pltpu.VMEM((CHUNK, HIST, DIM), jnp.float32),
            pltpu.VMEM((CHUNK, DIM), jnp.float32),
            pltpu.VMEM((1, DIM), jnp.float32),
            pltpu.SemaphoreType.DMA,
        ],
        compiler_params=pltpu.CompilerParams(use_tc_tiling_on_sc=False,
                                             needs_layout_passes=False),
    )
    return ker(ids, table)


def kernel(code_ids_batch, table):
    ids = code_ids_batch.astype(jnp.int32)
    return _visit_encode(ids, table)
